# baseline (speedup 1.0000x reference)
import functools

import jax
import jax.numpy as jnp
from jax.experimental import pallas as pl
from jax.experimental.pallas import tpu as pltpu

LAMBDA_COORD = 5.0
LAMBDA_NOOBJ = 0.5

BN = 7680


def _safe_log(x):
    return jnp.clip(jnp.log(x), -100.0, None)


def _match_kernel(ps_ref, bb_ref, idx_ref, out_ref, best_iou_ref,
                  best_idx_ref, w5_ref, sacc_ref, *, n_total, nb_count):
    nb = pl.program_id(1)

    ps = ps_ref[0]
    px = ps[0:1, :]
    py = ps[1:2, :]
    pw = ps[2:3, :]
    ph = ps[3:4, :]
    conf = ps[4:5, :]

    bb = bb_ref[0]
    x1 = bb[:, 0:1]
    y1 = bb[:, 1:2]
    x2 = bb[:, 2:3]
    y2 = bb[:, 3:4]
    gx = (x1 + x2) / 2
    gy = (y1 + y2) / 2
    gw = x2 - x1
    gh = y2 - y1

    b1_x1 = px - pw / 2
    b1_y1 = py - ph / 2
    b1_x2 = px + pw / 2
    b1_y2 = py + ph / 2
    b2_x1 = gx - gw / 2
    b2_y1 = gy - gh / 2
    b2_x2 = gx + gw / 2
    b2_y2 = gy + gh / 2
    ix1 = jnp.maximum(b1_x1, b2_x1)
    iy1 = jnp.maximum(b1_y1, b2_y1)
    ix2 = jnp.minimum(b1_x2, b2_x2)
    iy2 = jnp.minimum(b1_y2, b2_y2)
    inter = jnp.clip(ix2 - ix1, 0.0, None) * jnp.clip(iy2 - iy1, 0.0, None)
    a1 = (b1_x2 - b1_x1) * (b1_y2 - b1_y1)
    a2 = (b2_x2 - b2_x1) * (b2_y2 - b2_y1)
    union = a1 + a2 - inter
    iou = inter / (union + 1e-16)

    lane = jax.lax.broadcasted_iota(jnp.int32, (1, BN), 1)
    grow = lane + nb * BN
    valid = grow < n_total
    iou = jnp.where(valid, iou, -jnp.inf)

    blockmax = jnp.max(iou, axis=1, keepdims=True)
    big = jnp.int32(2**30)
    idxmat = jnp.where(iou == blockmax, grow, big)
    argg = jnp.min(idxmat, axis=1, keepdims=True)
    argl = argg - nb * BN

    prev_best = jnp.where(nb == 0, -jnp.inf, best_iou_ref[...])
    prev_idx = jnp.where(nb == 0, 0, best_idx_ref[...])
    upd = blockmax > prev_best
    best_iou_ref[...] = jnp.where(upd, blockmax, prev_best)
    best_idx_ref[...] = jnp.where(upd, argg, prev_idx)

    onehot = (lane == argl).astype(jnp.float32)
    psm = jnp.where(valid, ps, 0.0)
    w5_block = jax.lax.dot_general(
        onehot, psm, (((1,), (1,)), ((), ())),
        preferred_element_type=jnp.float32)
    prev_w5 = jnp.where(nb == 0, 0.0, w5_ref[...])
    w5_ref[...] = jnp.where(upd, w5_block, prev_w5)

    l1m = jnp.clip(jnp.log(1.0 - conf), -100.0, None)
    l1m = jnp.where(valid, l1m, 0.0)
    s_part = jnp.sum(l1m)
    prev_s = jnp.where(nb == 0, 0.0, sacc_ref[0, 0])
    sacc_ref[0, 0] = prev_s + s_part

    @pl.when(nb == nb_count - 1)
    def _emit():
        idx_ref[0] = best_idx_ref[...]
        w5 = w5_ref[...]
        pb = w5[:, 0:4]
        conf_b = w5[:, 4:5]
        gt = jnp.concatenate([gx, gy, gw, gh], axis=1)
        loss_coord = LAMBDA_COORD * jnp.sum((pb - gt) ** 2)
        loss_conf = jnp.sum(-_safe_log(conf_b))
        s = sacc_ref[0, 0]
        l1m_best = jnp.clip(jnp.log(1.0 - conf_b), -100.0, None)
        loss_noobj = LAMBDA_NOOBJ * jnp.sum(-(s - l1m_best))
        out_ref[0, 0, 0] = loss_coord + loss_conf + loss_noobj


def _cls_kernel(idx_ref, *refs, n_gt, n_cls):
    row_refs = refs[:n_gt]
    cls_ref = refs[n_gt]
    out_ref = refs[n_gt + 1]
    rows = jnp.concatenate([r[0, 0] for r in row_refs], axis=0)
    cls_b = rows[:, 5:]
    cid = cls_ref[0]
    ccol = jax.lax.broadcasted_iota(jnp.int32, (n_gt, n_cls), 1)
    oh = (ccol == cid).astype(jnp.float32)
    out_ref[0, 0, 0] = jnp.sum(
        -(oh * _safe_log(cls_b) + (1.0 - oh) * _safe_log(1.0 - cls_b)))


def kernel(pred, bboxes, classes):
    B, N, D = pred.shape
    G = bboxes.shape[1]
    C = D - 5
    nb_count = pl.cdiv(N, BN)
    ps = jnp.transpose(pred[:, :, :5], (0, 2, 1))
    cls3 = classes.reshape(B, G, 1)

    best_idx, out1 = pl.pallas_call(
        functools.partial(_match_kernel, n_total=N, nb_count=nb_count),
        grid=(B, nb_count),
        in_specs=[
            pl.BlockSpec((1, 5, BN), lambda b, nb: (b, 0, nb)),
            pl.BlockSpec((1, G, 4), lambda b, nb: (b, 0, 0)),
        ],
        out_specs=[
            pl.BlockSpec((1, G, 1), lambda b, nb: (b, 0, 0)),
            pl.BlockSpec((1, 1, 1), lambda b, nb: (b, 0, 0),
                         memory_space=pltpu.SMEM),
        ],
        out_shape=[
            jax.ShapeDtypeStruct((B, G, 1), jnp.int32),
            jax.ShapeDtypeStruct((B, 1, 1), jnp.float32),
        ],
        scratch_shapes=[
            pltpu.VMEM((G, 1), jnp.float32),
            pltpu.VMEM((G, 1), jnp.int32),
            pltpu.VMEM((G, 5), jnp.float32),
            pltpu.SMEM((1, 1), jnp.float32),
        ],
        compiler_params=pltpu.CompilerParams(
            dimension_semantics=("arbitrary", "arbitrary")),
    )(ps, bboxes)

    pred4 = pred.reshape(B, N, 1, D)
    idx2d = best_idx.reshape(B, G)

    def _row_spec(i):
        return pl.BlockSpec(
            (1, 1, 1, D), lambda b, idx_ref, i=i: (b, idx_ref[b, i], 0, 0))

    out2 = pl.pallas_call(
        functools.partial(_cls_kernel, n_gt=G, n_cls=C),
        grid_spec=pltpu.PrefetchScalarGridSpec(
            num_scalar_prefetch=1,
            grid=(B,),
            in_specs=[_row_spec(i) for i in range(G)] + [
                pl.BlockSpec((1, G, 1), lambda b, idx_ref: (b, 0, 0)),
            ],
            out_specs=pl.BlockSpec((1, 1, 1), lambda b, idx_ref: (b, 0, 0),
                                   memory_space=pltpu.SMEM),
        ),
        out_shape=jax.ShapeDtypeStruct((B, 1, 1), jnp.float32),
        compiler_params=pltpu.CompilerParams(
            dimension_semantics=("arbitrary",)),
    )(idx2d, *([pred4] * G), cls3)
    return (jnp.sum(out1) + jnp.sum(out2)) / B


# speedup vs baseline: 1.3285x; 1.3285x over previous
import functools

import jax
import jax.numpy as jnp
from jax.experimental import pallas as pl
from jax.experimental.pallas import tpu as pltpu

LAMBDA_COORD = 5.0
LAMBDA_NOOBJ = 0.5

BN = 7680


def _safe_log(x):
    return jnp.clip(jnp.log(x), -100.0, None)


def _match_kernel(ps_ref, bb_ref, idx_ref, out_ref, best_iou_ref,
                  best_idx_ref, w5_ref, sacc_ref, *, n_total, nb_count):
    nb = pl.program_id(1)

    ps = ps_ref[0]
    px = ps[0:1, :]
    py = ps[1:2, :]
    pw = ps[2:3, :]
    ph = ps[3:4, :]
    conf = ps[4:5, :]

    bb = bb_ref[0]
    x1 = bb[:, 0:1]
    y1 = bb[:, 1:2]
    x2 = bb[:, 2:3]
    y2 = bb[:, 3:4]
    gx = (x1 + x2) / 2
    gy = (y1 + y2) / 2
    gw = x2 - x1
    gh = y2 - y1

    b1_x1 = px - pw / 2
    b1_y1 = py - ph / 2
    b1_x2 = px + pw / 2
    b1_y2 = py + ph / 2
    b2_x1 = gx - gw / 2
    b2_y1 = gy - gh / 2
    b2_x2 = gx + gw / 2
    b2_y2 = gy + gh / 2
    ix1 = jnp.maximum(b1_x1, b2_x1)
    iy1 = jnp.maximum(b1_y1, b2_y1)
    ix2 = jnp.minimum(b1_x2, b2_x2)
    iy2 = jnp.minimum(b1_y2, b2_y2)
    inter = jnp.clip(ix2 - ix1, 0.0, None) * jnp.clip(iy2 - iy1, 0.0, None)
    a1 = (b1_x2 - b1_x1) * (b1_y2 - b1_y1)
    a2 = (b2_x2 - b2_x1) * (b2_y2 - b2_y1)
    union = a1 + a2 - inter
    iou = inter / (union + 1e-16)

    lane = jax.lax.broadcasted_iota(jnp.int32, (1, BN), 1)
    grow = lane + nb * BN
    valid = grow < n_total
    iou = jnp.where(valid, iou, -jnp.inf)

    blockmax = jnp.max(iou, axis=1, keepdims=True)
    big = jnp.int32(2**30)
    idxmat = jnp.where(iou == blockmax, grow, big)
    argg = jnp.min(idxmat, axis=1, keepdims=True)
    argl = argg - nb * BN

    prev_best = jnp.where(nb == 0, -jnp.inf, best_iou_ref[...])
    prev_idx = jnp.where(nb == 0, 0, best_idx_ref[...])
    upd = blockmax > prev_best
    best_iou_ref[...] = jnp.where(upd, blockmax, prev_best)
    best_idx_ref[...] = jnp.where(upd, argg, prev_idx)

    onehot = (lane == argl).astype(jnp.float32)
    psm = jnp.where(valid, ps, 0.0)
    w5_block = jax.lax.dot_general(
        onehot, psm, (((1,), (1,)), ((), ())),
        preferred_element_type=jnp.float32)
    prev_w5 = jnp.where(nb == 0, 0.0, w5_ref[...])
    w5_ref[...] = jnp.where(upd, w5_block, prev_w5)

    l1m = jnp.clip(jnp.log(1.0 - conf), -100.0, None)
    l1m = jnp.where(valid, l1m, 0.0)
    s_part = jnp.sum(l1m)
    prev_s = jnp.where(nb == 0, 0.0, sacc_ref[0, 0])
    sacc_ref[0, 0] = prev_s + s_part

    @pl.when(nb == nb_count - 1)
    def _emit():
        idx_ref[0] = best_idx_ref[...]
        w5 = w5_ref[...]
        pb = w5[:, 0:4]
        conf_b = w5[:, 4:5]
        gt = jnp.concatenate([gx, gy, gw, gh], axis=1)
        loss_coord = LAMBDA_COORD * jnp.sum((pb - gt) ** 2)
        loss_conf = jnp.sum(-_safe_log(conf_b))
        s = sacc_ref[0, 0]
        l1m_best = jnp.clip(jnp.log(1.0 - conf_b), -100.0, None)
        loss_noobj = LAMBDA_NOOBJ * jnp.sum(-(s - l1m_best))
        out_ref[0, 0, 0] = loss_coord + loss_conf + loss_noobj


def _cls_kernel(idx_ref, *refs, n_gt, n_cls):
    b = pl.program_id(0)
    row_refs = refs[:n_gt]
    cls_ref = refs[n_gt]
    out_ref = refs[n_gt + 1]
    rows = jnp.concatenate(
        [r[0, pl.ds(idx_ref[b, i] % 8, 1), :]
         for i, r in enumerate(row_refs)], axis=0)
    cls_b = rows[:, 5:]
    cid = cls_ref[0]
    ccol = jax.lax.broadcasted_iota(jnp.int32, (n_gt, n_cls), 1)
    oh = (ccol == cid).astype(jnp.float32)
    out_ref[0, 0, 0] = jnp.sum(
        -(oh * _safe_log(cls_b) + (1.0 - oh) * _safe_log(1.0 - cls_b)))


def kernel(pred, bboxes, classes):
    B, N, D = pred.shape
    G = bboxes.shape[1]
    C = D - 5
    nb_count = pl.cdiv(N, BN)
    ps = jnp.transpose(pred[:, :, :5], (0, 2, 1))
    cls3 = classes.reshape(B, G, 1)

    best_idx, out1 = pl.pallas_call(
        functools.partial(_match_kernel, n_total=N, nb_count=nb_count),
        grid=(B, nb_count),
        in_specs=[
            pl.BlockSpec((1, 5, BN), lambda b, nb: (b, 0, nb)),
            pl.BlockSpec((1, G, 4), lambda b, nb: (b, 0, 0)),
        ],
        out_specs=[
            pl.BlockSpec((1, G, 1), lambda b, nb: (b, 0, 0)),
            pl.BlockSpec((1, 1, 1), lambda b, nb: (b, 0, 0),
                         memory_space=pltpu.SMEM),
        ],
        out_shape=[
            jax.ShapeDtypeStruct((B, G, 1), jnp.int32),
            jax.ShapeDtypeStruct((B, 1, 1), jnp.float32),
        ],
        scratch_shapes=[
            pltpu.VMEM((G, 1), jnp.float32),
            pltpu.VMEM((G, 1), jnp.int32),
            pltpu.VMEM((G, 5), jnp.float32),
            pltpu.SMEM((1, 1), jnp.float32),
        ],
        compiler_params=pltpu.CompilerParams(
            dimension_semantics=("arbitrary", "arbitrary")),
    )(ps, bboxes)

    idx2d = best_idx.reshape(B, G)

    def _row_spec(i):
        return pl.BlockSpec(
            (1, 8, D), lambda b, idx_ref, i=i: (b, idx_ref[b, i] // 8, 0))

    out2 = pl.pallas_call(
        functools.partial(_cls_kernel, n_gt=G, n_cls=C),
        grid_spec=pltpu.PrefetchScalarGridSpec(
            num_scalar_prefetch=1,
            grid=(B,),
            in_specs=[_row_spec(i) for i in range(G)] + [
                pl.BlockSpec((1, G, 1), lambda b, idx_ref: (b, 0, 0)),
            ],
            out_specs=pl.BlockSpec((1, 1, 1), lambda b, idx_ref: (b, 0, 0),
                                   memory_space=pltpu.SMEM),
        ),
        out_shape=jax.ShapeDtypeStruct((B, 1, 1), jnp.float32),
        compiler_params=pltpu.CompilerParams(
            dimension_semantics=("arbitrary",)),
    )(idx2d, *([pred] * G), cls3)
    return (jnp.sum(out1) + jnp.sum(out2)) / B


# speedup vs baseline: 1.3821x; 1.0403x over previous
import functools

import jax
import jax.numpy as jnp
from jax.experimental import pallas as pl
from jax.experimental.pallas import tpu as pltpu

LAMBDA_COORD = 5.0
LAMBDA_NOOBJ = 0.5

BN = 7680


def _safe_log(x):
    return jnp.clip(jnp.log(x), -100.0, None)


def _yolo_kernel(pred_ref, ps_ref, bb_ref, cls_ref, out_ref, best_iou_ref,
                 w_ref, s_ref, *, n_total, nb_count, n_cls, n_batch, n_gt):
    b = pl.program_id(0)
    nb = pl.program_id(1)

    block = pred_ref[0]
    ps = ps_ref[0]
    px = ps[0:1, :]
    py = ps[1:2, :]
    pw = ps[2:3, :]
    ph = ps[3:4, :]
    conf = ps[4:5, :]

    bb = bb_ref[0]
    x1 = bb[:, 0:1]
    y1 = bb[:, 1:2]
    x2 = bb[:, 2:3]
    y2 = bb[:, 3:4]
    gx = (x1 + x2) / 2
    gy = (y1 + y2) / 2
    gw = x2 - x1
    gh = y2 - y1

    b1_x1 = px - pw / 2
    b1_y1 = py - ph / 2
    b1_x2 = px + pw / 2
    b1_y2 = py + ph / 2
    b2_x1 = gx - gw / 2
    b2_y1 = gy - gh / 2
    b2_x2 = gx + gw / 2
    b2_y2 = gy + gh / 2
    ix1 = jnp.maximum(b1_x1, b2_x1)
    iy1 = jnp.maximum(b1_y1, b2_y1)
    ix2 = jnp.minimum(b1_x2, b2_x2)
    iy2 = jnp.minimum(b1_y2, b2_y2)
    inter = jnp.clip(ix2 - ix1, 0.0, None) * jnp.clip(iy2 - iy1, 0.0, None)
    a1 = (b1_x2 - b1_x1) * (b1_y2 - b1_y1)
    a2 = (b2_x2 - b2_x1) * (b2_y2 - b2_y1)
    union = a1 + a2 - inter
    iou = inter / (union + 1e-16)

    lane = jax.lax.broadcasted_iota(jnp.int32, (1, BN), 1)
    grow = lane + nb * BN
    valid = grow < n_total
    iou = jnp.where(valid, iou, -jnp.inf)

    blockmax = jnp.max(iou, axis=1, keepdims=True)
    big = jnp.int32(2**30)
    idxmat = jnp.where(iou == blockmax, grow, big)
    argg = jnp.min(idxmat, axis=1, keepdims=True)
    argl = argg - nb * BN

    prev_best = jnp.where(nb == 0, -jnp.inf, best_iou_ref[...])
    upd = blockmax > prev_best
    best_iou_ref[...] = jnp.where(upd, blockmax, prev_best)

    onehot = (lane == argl).astype(jnp.float32)

    def _update_w(blk):
        w_block = jax.lax.dot_general(
            onehot, blk, (((1,), (0,)), ((), ())),
            preferred_element_type=jnp.float32)
        prev_w = jnp.where(nb == 0, 0.0, w_ref[...])
        w_ref[...] = jnp.where(upd, w_block, prev_w)

    @pl.when(nb < nb_count - 1)
    def _full_block():
        _update_w(block)

    @pl.when(nb == nb_count - 1)
    def _tail_block():
        rowv = jax.lax.broadcasted_iota(jnp.int32, (BN, 1), 0) + nb * BN
        _update_w(jnp.where(rowv < n_total, block, 0.0))

    l1m = jnp.clip(jnp.log(1.0 - conf), -100.0, None)
    l1m = jnp.where(valid, l1m, 0.0)
    s_part = jnp.sum(l1m)
    prev_s = jnp.where(nb == 0, 0.0, s_ref[0, 0])
    s_ref[0, 0] = prev_s + s_part

    @pl.when(nb == nb_count - 1)
    def _finalize():
        w = w_ref[...]
        pb = w[:, 0:4]
        conf_b = w[:, 4:5]
        cls_b = w[:, 5:]
        gt = jnp.concatenate([gx, gy, gw, gh], axis=1)
        loss_coord = LAMBDA_COORD * jnp.sum((pb - gt) ** 2)
        loss_conf = jnp.sum(-_safe_log(conf_b))
        cid = cls_ref[0]
        ccol = jax.lax.broadcasted_iota(jnp.int32, (n_gt, n_cls), 1)
        oh = (ccol == cid).astype(jnp.float32)
        loss_cls = jnp.sum(
            -(oh * _safe_log(cls_b) + (1.0 - oh) * _safe_log(1.0 - cls_b)))
        s = s_ref[0, 0]
        l1m_best = jnp.clip(jnp.log(1.0 - conf_b), -100.0, None)
        loss_noobj = LAMBDA_NOOBJ * jnp.sum(-(s - l1m_best))
        out_ref[0, 0, 0] = loss_coord + loss_conf + loss_cls + loss_noobj


def kernel(pred, bboxes, classes):
    B, N, D = pred.shape
    G = bboxes.shape[1]
    C = D - 5
    nb_count = pl.cdiv(N, BN)
    cls3 = classes.reshape(B, G, 1)
    ps = jnp.transpose(pred[:, :, :5], (0, 2, 1))

    out = pl.pallas_call(
        functools.partial(_yolo_kernel, n_total=N, nb_count=nb_count,
                          n_cls=C, n_batch=B, n_gt=G),
        grid=(B, nb_count),
        in_specs=[
            pl.BlockSpec((1, BN, D), lambda b, nb: (b, nb, 0)),
            pl.BlockSpec((1, 5, BN), lambda b, nb: (b, 0, nb)),
            pl.BlockSpec((1, G, 4), lambda b, nb: (b, 0, 0)),
            pl.BlockSpec((1, G, 1), lambda b, nb: (b, 0, 0)),
        ],
        out_specs=pl.BlockSpec((1, 1, 1), lambda b, nb: (b, 0, 0),
                               memory_space=pltpu.SMEM),
        out_shape=jax.ShapeDtypeStruct((B, 1, 1), jnp.float32),
        scratch_shapes=[
            pltpu.VMEM((G, 1), jnp.float32),
            pltpu.VMEM((G, D), jnp.float32),
            pltpu.SMEM((1, 1), jnp.float32),
        ],
        compiler_params=pltpu.CompilerParams(
            dimension_semantics=("parallel", "arbitrary")),
    )(pred, ps, bboxes, cls3)
    return jnp.sum(out) / B


# speedup vs baseline: 1.5372x; 1.1122x over previous
import functools

import jax
import jax.numpy as jnp
from jax.experimental import pallas as pl
from jax.experimental.pallas import tpu as pltpu

LAMBDA_COORD = 5.0
LAMBDA_NOOBJ = 0.5

BN = 7584


def _safe_log(x):
    return jnp.clip(jnp.log(x), -100.0, None)


def _yolo_kernel(pred_ref, bb_ref, cls_ref, out_ref, best_iou_ref, w_ref,
                 s_ref, *, n_total, nb_count, n_cls, n_batch, n_gt):
    b = pl.program_id(0)
    nb = pl.program_id(1)

    block = pred_ref[0]
    bt = jnp.transpose(block[:, 0:8], (1, 0))
    px = bt[0:1, :]
    py = bt[1:2, :]
    pw = bt[2:3, :]
    ph = bt[3:4, :]
    conf = bt[4:5, :]

    bb = bb_ref[0]
    x1 = bb[:, 0:1]
    y1 = bb[:, 1:2]
    x2 = bb[:, 2:3]
    y2 = bb[:, 3:4]
    gx = (x1 + x2) / 2
    gy = (y1 + y2) / 2
    gw = x2 - x1
    gh = y2 - y1

    b1_x1 = px - pw / 2
    b1_y1 = py - ph / 2
    b1_x2 = px + pw / 2
    b1_y2 = py + ph / 2
    b2_x1 = gx - gw / 2
    b2_y1 = gy - gh / 2
    b2_x2 = gx + gw / 2
    b2_y2 = gy + gh / 2
    ix1 = jnp.maximum(b1_x1, b2_x1)
    iy1 = jnp.maximum(b1_y1, b2_y1)
    ix2 = jnp.minimum(b1_x2, b2_x2)
    iy2 = jnp.minimum(b1_y2, b2_y2)
    inter = jnp.clip(ix2 - ix1, 0.0, None) * jnp.clip(iy2 - iy1, 0.0, None)
    a1 = (b1_x2 - b1_x1) * (b1_y2 - b1_y1)
    a2 = (b2_x2 - b2_x1) * (b2_y2 - b2_y1)
    union = a1 + a2 - inter
    iou = inter / (union + 1e-16)

    lane = jax.lax.broadcasted_iota(jnp.int32, (1, BN), 1)
    grow = lane + nb * BN
    valid = grow < n_total
    iou = jnp.where(valid, iou, -jnp.inf)

    blockmax = jnp.max(iou, axis=1, keepdims=True)
    big = jnp.int32(2**30)
    idxmat = jnp.where(iou == blockmax, grow, big)
    argg = jnp.min(idxmat, axis=1, keepdims=True)
    argl = argg - nb * BN

    prev_best = jnp.where(nb == 0, -jnp.inf, best_iou_ref[...])
    upd = blockmax > prev_best
    best_iou_ref[...] = jnp.where(upd, blockmax, prev_best)

    onehot = (lane == argl).astype(jnp.float32)

    def _update_w(blk):
        w_block = jax.lax.dot_general(
            onehot, blk, (((1,), (0,)), ((), ())),
            preferred_element_type=jnp.float32)
        prev_w = jnp.where(nb == 0, 0.0, w_ref[...])
        w_ref[...] = jnp.where(upd, w_block, prev_w)

    @pl.when(nb < nb_count - 1)
    def _full_block():
        _update_w(block)

    @pl.when(nb == nb_count - 1)
    def _tail_block():
        rowv = jax.lax.broadcasted_iota(jnp.int32, (BN, 1), 0) + nb * BN
        _update_w(jnp.where(rowv < n_total, block, 0.0))

    l1m = jnp.clip(jnp.log(1.0 - conf), -100.0, None)
    l1m = jnp.where(valid, l1m, 0.0)
    s_part = jnp.sum(l1m)
    prev_s = jnp.where(nb == 0, 0.0, s_ref[0, 0])
    s_ref[0, 0] = prev_s + s_part

    @pl.when(nb == nb_count - 1)
    def _finalize():
        w = w_ref[...]
        pb = w[:, 0:4]
        conf_b = w[:, 4:5]
        cls_b = w[:, 5:]
        gt = jnp.concatenate([gx, gy, gw, gh], axis=1)
        loss_coord = LAMBDA_COORD * jnp.sum((pb - gt) ** 2)
        loss_conf = jnp.sum(-_safe_log(conf_b))
        cid = cls_ref[0]
        ccol = jax.lax.broadcasted_iota(jnp.int32, (n_gt, n_cls), 1)
        oh = (ccol == cid).astype(jnp.float32)
        loss_cls = jnp.sum(
            -(oh * _safe_log(cls_b) + (1.0 - oh) * _safe_log(1.0 - cls_b)))
        s = s_ref[0, 0]
        l1m_best = jnp.clip(jnp.log(1.0 - conf_b), -100.0, None)
        loss_noobj = LAMBDA_NOOBJ * jnp.sum(-(s - l1m_best))
        out_ref[0, 0, 0] = loss_coord + loss_conf + loss_cls + loss_noobj


def kernel(pred, bboxes, classes):
    B, N, D = pred.shape
    G = bboxes.shape[1]
    C = D - 5
    nb_count = pl.cdiv(N, BN)
    cls3 = classes.reshape(B, G, 1)

    out = pl.pallas_call(
        functools.partial(_yolo_kernel, n_total=N, nb_count=nb_count,
                          n_cls=C, n_batch=B, n_gt=G),
        grid=(B, nb_count),
        in_specs=[
            pl.BlockSpec((1, BN, D), lambda b, nb: (b, nb, 0)),
            pl.BlockSpec((1, G, 4), lambda b, nb: (b, 0, 0)),
            pl.BlockSpec((1, G, 1), lambda b, nb: (b, 0, 0)),
        ],
        out_specs=pl.BlockSpec((1, 1, 1), lambda b, nb: (b, 0, 0),
                               memory_space=pltpu.SMEM),
        out_shape=jax.ShapeDtypeStruct((B, 1, 1), jnp.float32),
        scratch_shapes=[
            pltpu.VMEM((G, 1), jnp.float32),
            pltpu.VMEM((G, D), jnp.float32),
            pltpu.SMEM((1, 1), jnp.float32),
        ],
        compiler_params=pltpu.CompilerParams(
            dimension_semantics=("parallel", "arbitrary")),
    )(pred, bboxes, cls3)
    return jnp.sum(out) / B


# speedup vs baseline: 1.5967x; 1.0387x over previous
import functools

import jax
import jax.numpy as jnp
from jax.experimental import pallas as pl
from jax.experimental.pallas import tpu as pltpu

LAMBDA_COORD = 5.0
LAMBDA_NOOBJ = 0.5

BN = 11392


def _safe_log(x):
    return jnp.clip(jnp.log(x), -100.0, None)


def _yolo_kernel(pred_ref, bb_ref, cls_ref, out_ref, best_iou_ref, w_ref,
                 s_ref, *, n_total, nb_count, n_cls, n_batch, n_gt):
    b = pl.program_id(0)
    nb = pl.program_id(1)

    block = pred_ref[0]
    bt = jnp.transpose(block[:, 0:8], (1, 0))
    px = bt[0:1, :]
    py = bt[1:2, :]
    pw = bt[2:3, :]
    ph = bt[3:4, :]
    conf = bt[4:5, :]

    bb = bb_ref[0]
    x1 = bb[:, 0:1]
    y1 = bb[:, 1:2]
    x2 = bb[:, 2:3]
    y2 = bb[:, 3:4]
    gx = (x1 + x2) / 2
    gy = (y1 + y2) / 2
    gw = x2 - x1
    gh = y2 - y1

    b1_x1 = px - pw / 2
    b1_y1 = py - ph / 2
    b1_x2 = px + pw / 2
    b1_y2 = py + ph / 2
    b2_x1 = gx - gw / 2
    b2_y1 = gy - gh / 2
    b2_x2 = gx + gw / 2
    b2_y2 = gy + gh / 2
    ix1 = jnp.maximum(b1_x1, b2_x1)
    iy1 = jnp.maximum(b1_y1, b2_y1)
    ix2 = jnp.minimum(b1_x2, b2_x2)
    iy2 = jnp.minimum(b1_y2, b2_y2)
    inter = jnp.clip(ix2 - ix1, 0.0, None) * jnp.clip(iy2 - iy1, 0.0, None)
    a1 = (b1_x2 - b1_x1) * (b1_y2 - b1_y1)
    a2 = (b2_x2 - b2_x1) * (b2_y2 - b2_y1)
    union = a1 + a2 - inter
    iou = inter / (union + 1e-16)

    lane = jax.lax.broadcasted_iota(jnp.int32, (1, BN), 1)
    grow = lane + nb * BN
    valid = grow < n_total
    iou = jnp.where(valid, iou, -jnp.inf)

    blockmax = jnp.max(iou, axis=1, keepdims=True)
    big = jnp.int32(2**30)
    idxmat = jnp.where(iou == blockmax, grow, big)
    argg = jnp.min(idxmat, axis=1, keepdims=True)
    argl = argg - nb * BN

    prev_best = jnp.where(nb == 0, -jnp.inf, best_iou_ref[...])
    upd = blockmax > prev_best
    best_iou_ref[...] = jnp.where(upd, blockmax, prev_best)

    onehot = (lane == argl).astype(jnp.float32)

    def _update_w(blk):
        w_block = jax.lax.dot_general(
            onehot, blk, (((1,), (0,)), ((), ())),
            preferred_element_type=jnp.float32)
        prev_w = jnp.where(nb == 0, 0.0, w_ref[...])
        w_ref[...] = jnp.where(upd, w_block, prev_w)

    @pl.when(nb < nb_count - 1)
    def _full_block():
        _update_w(block)

    @pl.when(nb == nb_count - 1)
    def _tail_block():
        rowv = jax.lax.broadcasted_iota(jnp.int32, (BN, 1), 0) + nb * BN
        _update_w(jnp.where(rowv < n_total, block, 0.0))

    l1m = jnp.clip(jnp.log(1.0 - conf), -100.0, None)
    l1m = jnp.where(valid, l1m, 0.0)
    s_part = jnp.sum(l1m)
    prev_s = jnp.where(nb == 0, 0.0, s_ref[0, 0])
    s_ref[0, 0] = prev_s + s_part

    @pl.when(nb == nb_count - 1)
    def _finalize():
        w = w_ref[...]
        pb = w[:, 0:4]
        conf_b = w[:, 4:5]
        cls_b = w[:, 5:]
        gt = jnp.concatenate([gx, gy, gw, gh], axis=1)
        loss_coord = LAMBDA_COORD * jnp.sum((pb - gt) ** 2)
        loss_conf = jnp.sum(-_safe_log(conf_b))
        cid = cls_ref[0]
        ccol = jax.lax.broadcasted_iota(jnp.int32, (n_gt, n_cls), 1)
        oh = (ccol == cid).astype(jnp.float32)
        loss_cls = jnp.sum(
            -(oh * _safe_log(cls_b) + (1.0 - oh) * _safe_log(1.0 - cls_b)))
        s = s_ref[0, 0]
        l1m_best = jnp.clip(jnp.log(1.0 - conf_b), -100.0, None)
        loss_noobj = LAMBDA_NOOBJ * jnp.sum(-(s - l1m_best))
        out_ref[0, 0, 0] = loss_coord + loss_conf + loss_cls + loss_noobj


def kernel(pred, bboxes, classes):
    B, N, D = pred.shape
    G = bboxes.shape[1]
    C = D - 5
    nb_count = pl.cdiv(N, BN)
    cls3 = classes.reshape(B, G, 1)

    out = pl.pallas_call(
        functools.partial(_yolo_kernel, n_total=N, nb_count=nb_count,
                          n_cls=C, n_batch=B, n_gt=G),
        grid=(B, nb_count),
        in_specs=[
            pl.BlockSpec((1, BN, D), lambda b, nb: (b, nb, 0)),
            pl.BlockSpec((1, G, 4), lambda b, nb: (b, 0, 0)),
            pl.BlockSpec((1, G, 1), lambda b, nb: (b, 0, 0)),
        ],
        out_specs=pl.BlockSpec((1, 1, 1), lambda b, nb: (b, 0, 0),
                               memory_space=pltpu.SMEM),
        out_shape=jax.ShapeDtypeStruct((B, 1, 1), jnp.float32),
        scratch_shapes=[
            pltpu.VMEM((G, 1), jnp.float32),
            pltpu.VMEM((G, D), jnp.float32),
            pltpu.SMEM((1, 1), jnp.float32),
        ],
        compiler_params=pltpu.CompilerParams(
            dimension_semantics=("parallel", "arbitrary")),
    )(pred, bboxes, cls3)
    return jnp.sum(out) / B


# speedup vs baseline: 1.7738x; 1.1110x over previous
import functools

import jax
import jax.numpy as jnp
from jax.experimental import pallas as pl
from jax.experimental.pallas import tpu as pltpu

LAMBDA_COORD = 5.0
LAMBDA_NOOBJ = 0.5

BN = 22784


def _safe_log(x):
    return jnp.clip(jnp.log(x), -100.0, None)


def _yolo_kernel(pred_ref, bb_ref, cls_ref, out_ref, best_iou_ref, w_ref,
                 s_ref, *, n_total, nb_count, n_cls, n_batch, n_gt):
    b = pl.program_id(0)
    nb = pl.program_id(1)

    block = pred_ref[0]
    bt = jnp.transpose(block[:, 0:8], (1, 0))
    px = bt[0:1, :]
    py = bt[1:2, :]
    pw = bt[2:3, :]
    ph = bt[3:4, :]
    conf = bt[4:5, :]

    bb = bb_ref[0]
    x1 = bb[:, 0:1]
    y1 = bb[:, 1:2]
    x2 = bb[:, 2:3]
    y2 = bb[:, 3:4]
    gx = (x1 + x2) / 2
    gy = (y1 + y2) / 2
    gw = x2 - x1
    gh = y2 - y1

    b1_x1 = px - pw / 2
    b1_y1 = py - ph / 2
    b1_x2 = px + pw / 2
    b1_y2 = py + ph / 2
    b2_x1 = gx - gw / 2
    b2_y1 = gy - gh / 2
    b2_x2 = gx + gw / 2
    b2_y2 = gy + gh / 2
    ix1 = jnp.maximum(b1_x1, b2_x1)
    iy1 = jnp.maximum(b1_y1, b2_y1)
    ix2 = jnp.minimum(b1_x2, b2_x2)
    iy2 = jnp.minimum(b1_y2, b2_y2)
    inter = jnp.clip(ix2 - ix1, 0.0, None) * jnp.clip(iy2 - iy1, 0.0, None)
    a1 = (b1_x2 - b1_x1) * (b1_y2 - b1_y1)
    a2 = (b2_x2 - b2_x1) * (b2_y2 - b2_y1)
    union = a1 + a2 - inter
    iou = inter / (union + 1e-16)

    lane = jax.lax.broadcasted_iota(jnp.int32, (1, BN), 1)
    grow = lane + nb * BN
    valid = grow < n_total
    iou = jnp.where(valid, iou, -jnp.inf)

    blockmax = jnp.max(iou, axis=1, keepdims=True)
    big = jnp.int32(2**30)
    idxmat = jnp.where(iou == blockmax, grow, big)
    argg = jnp.min(idxmat, axis=1, keepdims=True)
    argl = argg - nb * BN

    prev_best = jnp.where(nb == 0, -jnp.inf, best_iou_ref[...])
    upd = blockmax > prev_best
    best_iou_ref[...] = jnp.where(upd, blockmax, prev_best)

    onehot = (lane == argl).astype(jnp.float32)

    def _update_w(blk):
        w_block = jax.lax.dot_general(
            onehot, blk, (((1,), (0,)), ((), ())),
            preferred_element_type=jnp.float32)
        prev_w = jnp.where(nb == 0, 0.0, w_ref[...])
        w_ref[...] = jnp.where(upd, w_block, prev_w)

    @pl.when(nb < nb_count - 1)
    def _full_block():
        _update_w(block)

    @pl.when(nb == nb_count - 1)
    def _tail_block():
        rowv = jax.lax.broadcasted_iota(jnp.int32, (BN, 1), 0) + nb * BN
        _update_w(jnp.where(rowv < n_total, block, 0.0))

    l1m = jnp.clip(jnp.log(1.0 - conf), -100.0, None)
    l1m = jnp.where(valid, l1m, 0.0)
    s_part = jnp.sum(l1m)
    prev_s = jnp.where(nb == 0, 0.0, s_ref[0, 0])
    s_ref[0, 0] = prev_s + s_part

    @pl.when(nb == nb_count - 1)
    def _finalize():
        w = w_ref[...]
        pb = w[:, 0:4]
        conf_b = w[:, 4:5]
        cls_b = w[:, 5:]
        gt = jnp.concatenate([gx, gy, gw, gh], axis=1)
        loss_coord = LAMBDA_COORD * jnp.sum((pb - gt) ** 2)
        loss_conf = jnp.sum(-_safe_log(conf_b))
        cid = cls_ref[0]
        ccol = jax.lax.broadcasted_iota(jnp.int32, (n_gt, n_cls), 1)
        oh = (ccol == cid).astype(jnp.float32)
        loss_cls = jnp.sum(
            -(oh * _safe_log(cls_b) + (1.0 - oh) * _safe_log(1.0 - cls_b)))
        s = s_ref[0, 0]
        l1m_best = jnp.clip(jnp.log(1.0 - conf_b), -100.0, None)
        loss_noobj = LAMBDA_NOOBJ * jnp.sum(-(s - l1m_best))
        out_ref[0, 0, 0] = loss_coord + loss_conf + loss_cls + loss_noobj


def kernel(pred, bboxes, classes):
    B, N, D = pred.shape
    G = bboxes.shape[1]
    C = D - 5
    nb_count = pl.cdiv(N, BN)
    cls3 = classes.reshape(B, G, 1)

    out = pl.pallas_call(
        functools.partial(_yolo_kernel, n_total=N, nb_count=nb_count,
                          n_cls=C, n_batch=B, n_gt=G),
        grid=(B, nb_count),
        in_specs=[
            pl.BlockSpec((1, BN, D), lambda b, nb: (b, nb, 0)),
            pl.BlockSpec((1, G, 4), lambda b, nb: (b, 0, 0)),
            pl.BlockSpec((1, G, 1), lambda b, nb: (b, 0, 0)),
        ],
        out_specs=pl.BlockSpec((1, 1, 1), lambda b, nb: (b, 0, 0),
                               memory_space=pltpu.SMEM),
        out_shape=jax.ShapeDtypeStruct((B, 1, 1), jnp.float32),
        scratch_shapes=[
            pltpu.VMEM((G, 1), jnp.float32),
            pltpu.VMEM((G, D), jnp.float32),
            pltpu.SMEM((1, 1), jnp.float32),
        ],
        compiler_params=pltpu.CompilerParams(
            dimension_semantics=("parallel", "arbitrary")),
    )(pred, bboxes, cls3)
    return jnp.sum(out) / B
